# baseline (speedup 1.0000x reference)
import functools

import jax
import jax.numpy as jnp
from jax import lax
from jax.experimental import pallas as pl
from jax.experimental.pallas import tpu as pltpu
from jax.experimental.pallas import tpu_sc as plsc

B = 16384
V = 1000000
D = 64
NBASIS = 16

NC = 2
NS = 16
NW = NC * NS
BPW = B // NW
NCHUNK = 4
CHUNK = BPW // NCHUNK

BBLK = 2048
GRID = B // BBLK


def _sc_gather(tok3, emb2, lin_flat):
    mesh = plsc.VectorSubcoreMesh(core_axis_name="c", subcore_axis_name="s")

    @functools.partial(
        pl.kernel,
        mesh=mesh,
        out_type=[
            jax.ShapeDtypeStruct((B, 2 * D), jnp.float32),
            jax.ShapeDtypeStruct((B,), jnp.float32),
        ],
        scratch_types=[
            pltpu.VMEM((NCHUNK, CHUNK), jnp.int32),
            pltpu.VMEM((NCHUNK, CHUNK), jnp.int32),
            pltpu.VMEM((BPW, 2 * D), jnp.float32),
            pltpu.VMEM((BPW,), jnp.float32),
            pltpu.SemaphoreType.DMA,
            pltpu.SemaphoreType.DMA,
        ],
        compiler_params=pltpu.CompilerParams(use_tc_tiling_on_sc=True),
    )
    def k(tok_hbm, emb_hbm, lin_hbm, demb_hbm, dlin_hbm,
          idx_v, half_v, rows_v, lin_v, sem_e, sem_l):
        wid = lax.axis_index("s") * NC + lax.axis_index("c")
        base = wid * BPW
        pltpu.sync_copy(tok_hbm.at[wid], idx_v)
        for j in range(NCHUNK):
            for g in range(CHUNK // 16):
                sl = pl.ds(g * 16, 16)
                half_v.at[j][sl] = lax.shift_right_logical(idx_v.at[j][sl], 1)
        copies = []
        for j in range(NCHUNK):
            copies.append(
                pltpu.async_copy(
                    emb_hbm.at[half_v.at[j]],
                    rows_v.at[pl.ds(j * CHUNK, CHUNK)],
                    sem_e,
                )
            )
            copies.append(
                pltpu.async_copy(
                    lin_hbm.at[idx_v.at[j]],
                    lin_v.at[pl.ds(j * CHUNK, CHUNK)],
                    sem_l,
                )
            )
        for c in copies:
            c.wait()
        pltpu.sync_copy(rows_v, demb_hbm.at[pl.ds(base, BPW)])
        pltpu.sync_copy(lin_v, dlin_hbm.at[pl.ds(base, BPW)])

    return k(tok3, emb2, lin_flat)


def _tc_body(basis_ref, bt_ref, be_ref, bl_ref, mcol_ref, pcol_ref, mlane_ref,
             dlin_ref, demb_ref, emb_out, lin_out):
    cont = jnp.dot(basis_ref[...], be_ref[...], preferred_element_type=jnp.float32)
    d2 = demb_ref[...]
    podd = pcol_ref[...] > 0.0
    disc = jnp.where(podd, d2[:, D:], d2[:, :D])
    mrow = mcol_ref[...] > 0.0
    emb_out[...] = jnp.where(mrow, cont, disc)
    cont_lin = jnp.sum(bt_ref[...] * bl_ref[...], axis=0)
    mlane = mlane_ref[0, 0, :] > 0.0
    lin_out[0, 0, :] = jnp.where(mlane, cont_lin, dlin_ref[0, 0, :])


def _tc_combine(basis, basis_t, be, bl2, mask_col, par_col, mask_lane, dlin3, demb2):
    return pl.pallas_call(
        _tc_body,
        grid=(GRID,),
        in_specs=[
            pl.BlockSpec((BBLK, NBASIS), lambda i: (i, 0)),
            pl.BlockSpec((NBASIS, BBLK), lambda i: (0, i)),
            pl.BlockSpec((NBASIS, D), lambda i: (0, 0)),
            pl.BlockSpec((NBASIS, 1), lambda i: (0, 0)),
            pl.BlockSpec((BBLK, 1), lambda i: (i, 0)),
            pl.BlockSpec((BBLK, 1), lambda i: (i, 0)),
            pl.BlockSpec((1, 1, BBLK), lambda i: (i, 0, 0)),
            pl.BlockSpec((1, 1, BBLK), lambda i: (i, 0, 0)),
            pl.BlockSpec((BBLK, 2 * D), lambda i: (i, 0)),
        ],
        out_specs=[
            pl.BlockSpec((BBLK, D), lambda i: (i, 0)),
            pl.BlockSpec((1, 1, BBLK), lambda i: (i, 0, 0)),
        ],
        out_shape=[
            jax.ShapeDtypeStruct((B, D), jnp.float32),
            jax.ShapeDtypeStruct((GRID, 1, BBLK), jnp.float32),
        ],
    )(basis, basis_t, be, bl2, mask_col, par_col, mask_lane, dlin3, demb2)


def kernel(token_ids, positive_mask, basis, emb_table, lin_table, basis_embedding, basis_linear):
    tok = token_ids.astype(jnp.int32)
    tok3 = tok.reshape(NW, NCHUNK, CHUNK)
    emb2 = emb_table.reshape(V // 2, 2 * D)
    lin_flat = lin_table.reshape(V)
    demb2, dlin = _sc_gather(tok3, emb2, lin_flat)

    maskf = positive_mask.astype(jnp.float32)
    mask_col = maskf.reshape(B, 1)
    mask_lane = maskf.reshape(GRID, 1, BBLK)
    par_col = (tok & 1).astype(jnp.float32).reshape(B, 1)
    dlin3 = dlin.reshape(GRID, 1, BBLK)
    basis_t = basis.T
    bl2 = basis_linear.reshape(NBASIS, 1)

    emb, lin3 = _tc_combine(basis, basis_t, basis_embedding, bl2,
                            mask_col, par_col, mask_lane, dlin3, demb2)
    return emb, lin3.reshape(B)


# speedup vs baseline: 1.8441x; 1.8441x over previous
import functools

import jax
import jax.numpy as jnp
from jax import lax
from jax.experimental import pallas as pl
from jax.experimental.pallas import tpu as pltpu
from jax.experimental.pallas import tpu_sc as plsc

B = 16384
V = 1000000
D = 64
NBASIS = 16

NC = 2
NS = 16
NW = NC * NS
BPW = B // NW
NCHUNK = 4
CHUNK = BPW // NCHUNK

BBLK = 2048
GRID = B // BBLK

RB = 4096
RPACK = 503808
OPACK = 499712
TGRID = RPACK // RB
OFFB = OPACK // RB


def _tp_body(a_ref, b_ref, out_ref):
    out_ref[:, :D] = a_ref[...].T
    out_ref[:, D:] = b_ref[...].T


def _tc_transpose(emb_t):
    return pl.pallas_call(
        _tp_body,
        grid=(TGRID,),
        in_specs=[
            pl.BlockSpec((D, RB), lambda i: (0, i)),
            pl.BlockSpec((D, RB), lambda i: (0, i + OFFB)),
        ],
        out_specs=pl.BlockSpec((RB, 2 * D), lambda i: (i, 0)),
        out_shape=jax.ShapeDtypeStruct((RPACK, 2 * D), jnp.float32),
    )(emb_t, emb_t)


def _sc_gather(half3, tok3, emb2, lin_flat):
    mesh = plsc.VectorSubcoreMesh(core_axis_name="c", subcore_axis_name="s")

    @functools.partial(
        pl.kernel,
        mesh=mesh,
        out_type=[
            jax.ShapeDtypeStruct((B, 2 * D), jnp.float32),
            jax.ShapeDtypeStruct((B,), jnp.float32),
        ],
        scratch_types=[
            pltpu.VMEM((NCHUNK, CHUNK), jnp.int32),
            pltpu.VMEM((NCHUNK, CHUNK), jnp.int32),
            pltpu.VMEM((BPW, 2 * D), jnp.float32),
            pltpu.VMEM((BPW,), jnp.float32),
            pltpu.SemaphoreType.DMA,
            pltpu.SemaphoreType.DMA,
        ],
        compiler_params=pltpu.CompilerParams(use_tc_tiling_on_sc=True),
    )
    def k(half_hbm, tok_hbm, emb_hbm, lin_hbm, demb_hbm, dlin_hbm,
          hidx_v, tidx_v, rows_v, lin_v, sem_e, sem_l):
        wid = lax.axis_index("s") * NC + lax.axis_index("c")
        base = wid * BPW
        pltpu.sync_copy(half_hbm.at[wid], hidx_v)
        pltpu.sync_copy(tok_hbm.at[wid], tidx_v)
        copies = []
        for j in range(NCHUNK):
            copies.append(
                pltpu.async_copy(
                    emb_hbm.at[hidx_v.at[j]],
                    rows_v.at[pl.ds(j * CHUNK, CHUNK)],
                    sem_e,
                )
            )
            copies.append(
                pltpu.async_copy(
                    lin_hbm.at[tidx_v.at[j]],
                    lin_v.at[pl.ds(j * CHUNK, CHUNK)],
                    sem_l,
                )
            )
        for c in copies:
            c.wait()
        pltpu.sync_copy(rows_v, demb_hbm.at[pl.ds(base, BPW)])
        pltpu.sync_copy(lin_v, dlin_hbm.at[pl.ds(base, BPW)])

    return k(half3, tok3, emb2, lin_flat)


def _tc_body(basis_ref, bt_ref, be_ref, bl_ref, mcol_ref, pcol_ref, mlane_ref,
             dlin_ref, demb_ref, emb_out, lin_out):
    cont = jnp.dot(basis_ref[...], be_ref[...], preferred_element_type=jnp.float32)
    d2 = demb_ref[...]
    pupper = pcol_ref[...] > 0.0
    disc = jnp.where(pupper, d2[:, D:], d2[:, :D])
    mrow = mcol_ref[...] > 0.0
    emb_out[...] = jnp.where(mrow, cont, disc)
    cont_lin = jnp.sum(bt_ref[...] * bl_ref[...], axis=0)
    mlane = mlane_ref[0, 0, :] > 0.0
    lin_out[0, 0, :] = jnp.where(mlane, cont_lin, dlin_ref[0, 0, :])


def _tc_combine(basis, basis_t, be, bl2, mask_col, par_col, mask_lane, dlin3, demb2):
    return pl.pallas_call(
        _tc_body,
        grid=(GRID,),
        in_specs=[
            pl.BlockSpec((BBLK, NBASIS), lambda i: (i, 0)),
            pl.BlockSpec((NBASIS, BBLK), lambda i: (0, i)),
            pl.BlockSpec((NBASIS, D), lambda i: (0, 0)),
            pl.BlockSpec((NBASIS, 1), lambda i: (0, 0)),
            pl.BlockSpec((BBLK, 1), lambda i: (i, 0)),
            pl.BlockSpec((BBLK, 1), lambda i: (i, 0)),
            pl.BlockSpec((1, 1, BBLK), lambda i: (i, 0, 0)),
            pl.BlockSpec((1, 1, BBLK), lambda i: (i, 0, 0)),
            pl.BlockSpec((BBLK, 2 * D), lambda i: (i, 0)),
        ],
        out_specs=[
            pl.BlockSpec((BBLK, D), lambda i: (i, 0)),
            pl.BlockSpec((1, 1, BBLK), lambda i: (i, 0, 0)),
        ],
        out_shape=[
            jax.ShapeDtypeStruct((B, D), jnp.float32),
            jax.ShapeDtypeStruct((GRID, 1, BBLK), jnp.float32),
        ],
    )(basis, basis_t, be, bl2, mask_col, par_col, mask_lane, dlin3, demb2)


def kernel(token_ids, positive_mask, basis, emb_table, lin_table, basis_embedding, basis_linear):
    tok = token_ids.astype(jnp.int32)
    upper = tok >= OPACK
    half = jnp.where(upper, tok - OPACK, tok)
    tok3 = tok.reshape(NW, NCHUNK, CHUNK)
    half3 = half.reshape(NW, NCHUNK, CHUNK)
    emb2 = _tc_transpose(emb_table.T)
    lin_flat = lin_table.reshape(V)
    demb2, dlin = _sc_gather(half3, tok3, emb2, lin_flat)

    maskf = positive_mask.astype(jnp.float32)
    mask_col = maskf.reshape(B, 1)
    mask_lane = maskf.reshape(GRID, 1, BBLK)
    par_col = upper.astype(jnp.float32).reshape(B, 1)
    dlin3 = dlin.reshape(GRID, 1, BBLK)
    basis_t = basis.T
    bl2 = basis_linear.reshape(NBASIS, 1)

    emb, lin3 = _tc_combine(basis, basis_t, basis_embedding, bl2,
                            mask_col, par_col, mask_lane, dlin3, demb2)
    return emb, lin3.reshape(B)
